# baseline (speedup 1.0000x reference)
import functools

import jax
import jax.numpy as jnp
from jax import lax
from jax.experimental import pallas as pl
from jax.experimental.pallas import tpu as pltpu
from jax.experimental.pallas import tpu_sc as plsc

NB = 128
NG = NB // 16
NCLS = 20
IOU_T = 0.4
SCORE_T = 0.05
MAXOUT = 100
REC = 784
O_BOX = 0
O_SC = 400
O_CLS = 528
O_NV = 656


def _argmax_slots(ms, lane):
    m = ms[0]
    for g in range(1, NG):
        m = jnp.maximum(m, ms[g])
    best = jnp.max(m)
    pv = jnp.where(ms[0] == best, lane, 2 * NB)
    for g in range(1, NG):
        pv = jnp.minimum(pv, jnp.where(ms[g] == best, lane + g * 16, 2 * NB))
    pos = jnp.min(pv)
    return best, pos


def _sc_body(pred_hbm, out_hbm,
             pred_v0, pred_v1, bxv, sv, lsl, lbx, ost,
             sem0, sem1):
    info = plsc.get_sparse_core_info()
    nw = info.num_cores * info.num_subcores
    ipw = 64 // nw
    wid = lax.axis_index("s") * info.num_cores + lax.axis_index("c")
    lane = lax.iota(jnp.int32, 16)
    zf = jnp.zeros((16,), jnp.float32)
    zi = jnp.zeros((16,), jnp.int32)
    m0 = lane == 0

    preds = (pred_v0, pred_v1)
    cps = [pltpu.async_copy(pred_hbm.at[wid * ipw + t], preds[t], sem)
           for t, sem in zip(range(ipw), (sem0, sem1))]

    for g in range(ipw * REC // 16):
        ost[pl.ds(g * 16, 16)] = zf

    for t in range(ipw):
        cps[t].wait()
        pred_v = preds[t]
        base = t * REC

        maxvs, conf1s, conf2s = [], [], []
        for g in range(4):
            cell = lane + g * 16
            ii = (cell // 7).astype(jnp.float32)
            jj = (cell % 7).astype(jnp.float32)
            p = [pred_v[pl.ds(k * 64 + g * 16, 16)] for k in range(30)]
            maxv = p[10]
            for c in range(1, NCLS):
                maxv = jnp.maximum(maxv, p[10 + c])
            maxvs.append(maxv)
            conf1s.append(p[4])
            conf2s.append(p[9])
            for off, sbase in ((0, 0), (5, 64)):
                cx = ii * 64.0 + p[off] * 64.0
                cy = jj * 64.0 + p[off + 1] * 64.0
                w = jnp.minimum(p[off + 2] * 448.0, 448.0)
                h = jnp.minimum(p[off + 3] * 448.0, 448.0)
                bxv[pl.ds(0 * NB + sbase + g * 16, 16)] = cy - h / 2.0
                bxv[pl.ds(1 * NB + sbase + g * 16, 16)] = cx - w / 2.0
                bxv[pl.ds(2 * NB + sbase + g * 16, 16)] = cy + h / 2.0
                bxv[pl.ds(3 * NB + sbase + g * 16, 16)] = cx + w / 2.0

        def score_body(c, _):
            for g in range(4):
                v = pred_v[pl.ds((10 + c) * 64 + g * 16, 16)]
                s1 = jnp.where(v == maxvs[g], v * conf1s[g], 0.0)
                sv[pl.ds(c * NB + g * 16, 16)] = jnp.where(
                    s1 > SCORE_T, s1, -1.0)
                s2 = jnp.where(v == maxvs[g], v * conf2s[g], 0.0)
                sv[pl.ds(c * NB + 64 + g * 16, 16)] = jnp.where(
                    s2 > SCORE_T, s2, -1.0)
            return _

        lax.fori_loop(0, NCLS, score_body, jnp.int32(0))

        neg1 = zf - 1.0
        for g in range(NG):
            lsl[pl.ds(g * 16, 16)] = neg1

        def class_body(c, K):
            ms0 = [sv[pl.ds(c * NB + g * 16, 16)] for g in range(NG)]
            best0, p0 = _argmax_slots(ms0, lane)

            def cond(carry):
                return carry[0] > 0.0

            def body(carry):
                best, pos, k = carry
                ms = [sv[pl.ds(c * NB + g * 16, 16)] for g in range(NG)]
                kc = jnp.minimum(k, NB - 1)
                av = jnp.where(m0, zf + best, zf + c.astype(jnp.float32))
                ai = jnp.where(m0, zi + kc, zi + NB + kc)
                plsc.store_scatter(lsl, [ai], av, mask=lane < 2)
                plsc.store_scatter(lbx, [zi + kc], zi + pos, mask=m0)
                bq = plsc.load_gather(
                    bxv, [zi + pos + NB * jnp.minimum(lane, 3)], mask=lane < 4)
                yb1 = zf + bq[0]
                xb1 = zf + bq[1]
                yb2 = zf + bq[2]
                xb2 = zf + bq[3]
                areab = (yb2 - yb1) * (xb2 - xb1)
                nms = []
                for g in range(NG):
                    ay1 = bxv[pl.ds(0 * NB + g * 16, 16)]
                    ax1 = bxv[pl.ds(1 * NB + g * 16, 16)]
                    ay2 = bxv[pl.ds(2 * NB + g * 16, 16)]
                    ax2 = bxv[pl.ds(3 * NB + g * 16, 16)]
                    ih = jnp.maximum(jnp.minimum(yb2, ay2) - jnp.maximum(yb1, ay1), 0.0)
                    iw = jnp.maximum(jnp.minimum(xb2, ax2) - jnp.maximum(xb1, ax1), 0.0)
                    inter = ih * iw
                    union = areab + (ay2 - ay1) * (ax2 - ax1) - inter
                    iou = jnp.where(union > 0.0, inter / union, 0.0)
                    nms.append(jnp.where(iou > IOU_T, -1.0, ms[g]))
                for g in range(NG):
                    sv[pl.ds(c * NB + g * 16, 16)] = nms[g]
                nbest, npos = _argmax_slots(nms, lane)
                return (nbest, npos, k + 1)

            out = lax.while_loop(cond, body, (best0, p0, K))
            return out[2]

        K = lax.fori_loop(0, NCLS, class_body, jnp.int32(0))

        kv = jnp.minimum(K, MAXOUT)

        def sel_cond(e):
            return e < kv

        def sel_body(e):
            ls = [lsl[pl.ds(g * 16, 16)] for g in range(NG)]
            best, pos = _argmax_slots(ls, lane)
            pv = zi + pos
            cls_s = plsc.load_gather(lsl, [pv + NB], mask=m0)[0]
            bi = plsc.load_gather(lbx, [pv], mask=m0)[0]
            bv = plsc.load_gather(
                bxv, [zi + bi + NB * jnp.minimum(lane, 3)], mask=lane < 4)
            plsc.store_scatter(ost, [zi + (base + O_BOX) + e * 4 + lane], bv,
                               mask=lane < 4)
            av = jnp.where(m0, zf + best, zf + cls_s)
            ai = jnp.where(m0, zi + (base + O_SC) + e, zi + (base + O_CLS) + e)
            plsc.store_scatter(ost, [ai], av, mask=lane < 2)
            plsc.store_scatter(lsl, [zi + pos], neg1, mask=m0)
            return e + 1

        lax.while_loop(sel_cond, sel_body, jnp.int32(0))

        plsc.store_scatter(ost, [zi + (base + O_NV)],
                           zf + kv.astype(jnp.float32), mask=m0)

    pltpu.sync_copy(ost, out_hbm.at[wid])


def kernel(pred):
    B = pred.shape[0]
    pt = jnp.transpose(pred.reshape(B, 49, 30), (0, 2, 1))
    pt = jnp.pad(pt, ((0, 0), (0, 0), (0, 15)))
    pflat = pt.reshape(B, 30 * 64)

    mesh = plsc.VectorSubcoreMesh(core_axis_name="c", subcore_axis_name="s")
    f = pl.kernel(
        _sc_body,
        out_type=[jax.ShapeDtypeStruct((32, 2 * REC), jnp.float32)],
        mesh=mesh,
        compiler_params=pltpu.CompilerParams(needs_layout_passes=False),
        scratch_types=[
            pltpu.VMEM((30 * 64,), jnp.float32),
            pltpu.VMEM((30 * 64,), jnp.float32),
            pltpu.VMEM((4 * NB,), jnp.float32),
            pltpu.VMEM((NCLS * NB,), jnp.float32),
            pltpu.VMEM((2 * NB,), jnp.float32),
            pltpu.VMEM((NB,), jnp.int32),
            pltpu.VMEM((2 * REC,), jnp.float32),
            pltpu.SemaphoreType.DMA,
            pltpu.SemaphoreType.DMA,
        ],
    )
    o = f(pflat)
    if isinstance(o, (tuple, list)):
        o = o[0]
    o = o.reshape(B, REC)
    boxes = o[:, O_BOX:O_BOX + 400].reshape(B, MAXOUT, 4)
    sc = o[:, O_SC:O_SC + MAXOUT]
    cls = o[:, O_CLS:O_CLS + MAXOUT]
    nv = o[:, O_NV].astype(jnp.int32)
    return (nv, boxes, sc, cls)


# speedup vs baseline: 1.0634x; 1.0634x over previous
import functools

import jax
import jax.numpy as jnp
from jax import lax
from jax.experimental import pallas as pl
from jax.experimental.pallas import tpu as pltpu
from jax.experimental.pallas import tpu_sc as plsc

NB = 128
NG = NB // 16
NCLS = 20
IOU_T = 0.4
SCORE_T = 0.05
MAXOUT = 100
REC = 784
O_BOX = 0
O_SC = 400
O_CLS = 528
O_NV = 656


def _argmax_slots(ms, lane):
    m = ms[0]
    for g in range(1, NG):
        m = jnp.maximum(m, ms[g])
    best = jnp.max(m)
    pv = jnp.where(ms[0] == best, lane, 2 * NB)
    for g in range(1, NG):
        pv = jnp.minimum(pv, jnp.where(ms[g] == best, lane + g * 16, 2 * NB))
    pos = jnp.min(pv)
    return best, pos


def _sc_body(pred_hbm, out_hbm,
             pred_v0, pred_v1, y1v, x1v, y2v, x2v, sv, lsl, lbx, ost,
             sem0, sem1):
    info = plsc.get_sparse_core_info()
    nw = info.num_cores * info.num_subcores
    ipw = 64 // nw
    wid = lax.axis_index("s") * info.num_cores + lax.axis_index("c")
    lane = lax.iota(jnp.int32, 16)
    zf = jnp.zeros((16,), jnp.float32)
    zi = jnp.zeros((16,), jnp.int32)
    m0 = lane == 0

    preds = (pred_v0, pred_v1)
    cps = [pltpu.async_copy(pred_hbm.at[wid * ipw + t], preds[t], sem)
           for t, sem in zip(range(ipw), (sem0, sem1))]

    for g in range(ipw * REC // 16):
        ost[pl.ds(g * 16, 16)] = zf

    for t in range(ipw):
        cps[t].wait()
        pred_v = preds[t]
        base = t * REC

        for g in range(4):
            cell = lane + g * 16
            ii = (cell // 7).astype(jnp.float32)
            jj = (cell % 7).astype(jnp.float32)
            p = [pred_v[pl.ds(k * 64 + g * 16, 16)] for k in range(30)]
            maxv = p[10]
            for c in range(1, NCLS):
                maxv = jnp.maximum(maxv, p[10 + c])
            for off, sbase in ((0, 0), (5, 64)):
                cx = ii * 64.0 + p[off] * 64.0
                cy = jj * 64.0 + p[off + 1] * 64.0
                w = jnp.minimum(p[off + 2] * 448.0, 448.0)
                h = jnp.minimum(p[off + 3] * 448.0, 448.0)
                sl = pl.ds(sbase + g * 16, 16)
                y1v[sl] = cy - h / 2.0
                x1v[sl] = cx - w / 2.0
                y2v[sl] = cy + h / 2.0
                x2v[sl] = cx + w / 2.0
                conf = p[off + 4]
                for c in range(NCLS):
                    v = p[10 + c]
                    s = jnp.where(v == maxv, v * conf, 0.0)
                    sv[pl.ds(c * NB + sbase + g * 16, 16)] = jnp.where(
                        s > SCORE_T, s, -1.0)

        neg1 = zf - 1.0
        for g in range(NG):
            lsl[pl.ds(g * 16, 16)] = neg1

        def class_body(c, K):
            ms0 = [sv[pl.ds(c * NB + g * 16, 16)] for g in range(NG)]
            best0, p0 = _argmax_slots(ms0, lane)

            def cond(carry):
                return carry[NG] > 0.0

            def body(carry):
                ms = list(carry[:NG])
                best, pos, k = carry[NG], carry[NG + 1], carry[NG + 2]
                kc = jnp.minimum(k, NB - 1)
                av = jnp.where(m0, zf + best, zf + c.astype(jnp.float32))
                ai = jnp.where(m0, zi + kc, zi + NB + kc)
                plsc.store_scatter(lsl, [ai], av, mask=lane < 2)
                plsc.store_scatter(lbx, [zi + kc], zi + pos, mask=m0)
                pv = zi + pos
                yb1 = plsc.load_gather(y1v, [pv])
                xb1 = plsc.load_gather(x1v, [pv])
                yb2 = plsc.load_gather(y2v, [pv])
                xb2 = plsc.load_gather(x2v, [pv])
                areab = (yb2 - yb1) * (xb2 - xb1)
                nms = []
                for g in range(NG):
                    sl = pl.ds(g * 16, 16)
                    ay1 = y1v[sl]
                    ax1 = x1v[sl]
                    ay2 = y2v[sl]
                    ax2 = x2v[sl]
                    ih = jnp.maximum(jnp.minimum(yb2, ay2) - jnp.maximum(yb1, ay1), 0.0)
                    iw = jnp.maximum(jnp.minimum(xb2, ax2) - jnp.maximum(xb1, ax1), 0.0)
                    inter = ih * iw
                    union = areab + (ay2 - ay1) * (ax2 - ax1) - inter
                    iou = jnp.where(union > 0.0, inter / union, 0.0)
                    nms.append(jnp.where(iou > IOU_T, -1.0, ms[g]))
                nbest, npos = _argmax_slots(nms, lane)
                return (*nms, nbest, npos, k + 1)

            out = lax.while_loop(cond, body, (*ms0, best0, p0, K))
            return out[NG + 2]

        K = lax.fori_loop(0, NCLS, class_body, jnp.int32(0))

        kv = jnp.minimum(K, MAXOUT)
        ls0 = [lsl[pl.ds(g * 16, 16)] for g in range(NG)]

        def sel_cond(carry):
            return carry[NG] < kv

        def sel_body(carry):
            ls = list(carry[:NG])
            e = carry[NG]
            best, pos = _argmax_slots(ls, lane)
            pv = zi + pos
            clsv = plsc.load_gather(lsl, [pv + NB])
            biv = plsc.load_gather(lbx, [pv])
            y1b = plsc.load_gather(y1v, [biv])
            x1b = plsc.load_gather(x1v, [biv])
            y2b = plsc.load_gather(y2v, [biv])
            x2b = plsc.load_gather(x2v, [biv])
            bv = jnp.where(lane == 0, y1b,
                           jnp.where(lane == 1, x1b,
                                     jnp.where(lane == 2, y2b, x2b)))
            plsc.store_scatter(ost, [zi + (base + O_BOX) + e * 4 + lane], bv,
                               mask=lane < 4)
            av = jnp.where(m0, zf + best, clsv)
            ai = jnp.where(m0, zi + (base + O_SC) + e, zi + (base + O_CLS) + e)
            plsc.store_scatter(ost, [ai], av, mask=lane < 2)
            nls = [jnp.where(lane + g * 16 == pos, -1.0, ls[g]) for g in range(NG)]
            return (*nls, e + 1)

        lax.while_loop(sel_cond, sel_body, (*ls0, jnp.int32(0)))

        plsc.store_scatter(ost, [zi + (base + O_NV)],
                           zf + kv.astype(jnp.float32), mask=m0)

    pltpu.sync_copy(ost, out_hbm.at[wid])


def kernel(pred):
    B = pred.shape[0]
    pt = jnp.transpose(pred.reshape(B, 49, 30), (0, 2, 1))
    pt = jnp.pad(pt, ((0, 0), (0, 0), (0, 15)))
    pflat = pt.reshape(B, 30 * 64)

    mesh = plsc.VectorSubcoreMesh(core_axis_name="c", subcore_axis_name="s")
    f = pl.kernel(
        _sc_body,
        out_type=[jax.ShapeDtypeStruct((32, 2 * REC), jnp.float32)],
        mesh=mesh,
        compiler_params=pltpu.CompilerParams(needs_layout_passes=False),
        scratch_types=[
            pltpu.VMEM((30 * 64,), jnp.float32),
            pltpu.VMEM((30 * 64,), jnp.float32),
            pltpu.VMEM((NB,), jnp.float32),
            pltpu.VMEM((NB,), jnp.float32),
            pltpu.VMEM((NB,), jnp.float32),
            pltpu.VMEM((NB,), jnp.float32),
            pltpu.VMEM((NCLS * NB,), jnp.float32),
            pltpu.VMEM((2 * NB,), jnp.float32),
            pltpu.VMEM((NB,), jnp.int32),
            pltpu.VMEM((2 * REC,), jnp.float32),
            pltpu.SemaphoreType.DMA,
            pltpu.SemaphoreType.DMA,
        ],
    )
    o = f(pflat)
    if isinstance(o, (tuple, list)):
        o = o[0]
    o = o.reshape(B, REC)
    boxes = o[:, O_BOX:O_BOX + 400].reshape(B, MAXOUT, 4)
    sc = o[:, O_SC:O_SC + MAXOUT]
    cls = o[:, O_CLS:O_CLS + MAXOUT]
    nv = o[:, O_NV].astype(jnp.int32)
    return (nv, boxes, sc, cls)
